# Initial kernel scaffold; baseline (speedup 1.0000x reference)
import jax
import jax.numpy as jnp
from jax.experimental import pallas as pl


def kernel(coordinates, atom_ix, weights, bias):
    raise NotImplementedError("write your pallas kernel here")


# speedup vs baseline: 5744.4708x; 5744.4708x over previous
import jax
import jax.numpy as jnp
from jax import lax
from jax.experimental import pallas as pl
from jax.experimental.pallas import tpu as pltpu
from jax.experimental.pallas import tpu_sc as plsc

N_ATOMS = 4096
N_ELEM = 118
W_LEN = N_ELEM * (N_ELEM + 1) // 2
W_PAD = 7040
COULOMB = -231000.0
L = 16


def _sc_body(xs_hbm, ai_hbm, w_hbm, bias_hbm, out_hbm,
             xs_v, ai_v, w_v, cnt_v, bias_v, out_v):
    cid = lax.axis_index("c")
    sid = lax.axis_index("s")

    @pl.when(jnp.logical_and(cid == 0, sid == 0))
    def _():
        pltpu.sync_copy(xs_hbm, xs_v)
        pltpu.sync_copy(ai_hbm, ai_v)
        pltpu.sync_copy(w_hbm, w_v)
        pltpu.sync_copy(bias_hbm, bias_v)

        zero16 = jnp.zeros((L,), jnp.float32)
        ones16 = jnp.ones((L,), jnp.float32)
        iota = lax.iota(jnp.int32, L)

        for k in range(8):
            cnt_v[pl.ds(k * L, L)] = zero16

        def hist_body(i, carry):
            idx = ai_v[pl.ds(i * L, L)]
            plsc.addupdate_scatter(cnt_v, [idx], ones16)
            return carry
        lax.fori_loop(0, N_ATOMS // L, hist_body, 0)

        def comp_sums(c0):
            def body(i, carry):
                s1, sv = carry
                v = xs_v[pl.ds(c0 * N_ATOMS + i * L, L)]
                return (s1 + v * v, sv + v)
            return lax.fori_loop(0, N_ATOMS // L, body, (zero16, zero16))

        s1x, svx = comp_sums(0)
        s1y, svy = comp_sums(1)
        s1z, svz = comp_sums(2)
        s1 = jnp.sum(s1x + s1y + s1z)
        sx = jnp.sum(svx)
        sy = jnp.sum(svy)
        sz = jnp.sum(svz)
        d2 = 2.0 * N_ATOMS * s1 - 2.0 * (sx * sx + sy * sy + sz * sz)

        def w_body(a, wacc):
            t = a * (a + 1) // 2
            racc = zero16
            for k in range(8):
                idx = t + k * L + iota
                wv = plsc.load_gather(w_v, [idx])
                cb = cnt_v[pl.ds(k * L, L)]
                racc = racc + wv * cb
            cnta = plsc.load_gather(cnt_v, [jnp.full((L,), 0, jnp.int32) + a])
            return wacc + cnta * racc
        wacc = lax.fori_loop(0, N_ELEM, w_body, zero16)
        wsum = jnp.sum(wacc)

        d2v = jnp.full((L,), d2)
        bits = lax.bitcast_convert_type(d2v, jnp.int32)
        y = lax.bitcast_convert_type(jnp.int32(0x5F3759DF) - (bits >> 1),
                                     jnp.float32)
        for _ in range(4):
            y = y * (1.5 - 0.5 * d2v * y * y)
        rd = jnp.where(d2v > 0.0, y, jnp.float32(3.4028235e38))

        bv = bias_v[pl.ds(0, L)]
        out_v[pl.ds(0, L)] = COULOMB * wsum * rd + bv
        pltpu.sync_copy(out_v, out_hbm)


_mesh = plsc.VectorSubcoreMesh(core_axis_name="c", subcore_axis_name="s")

_sc_run = pl.kernel(
    _sc_body,
    out_type=jax.ShapeDtypeStruct((L,), jnp.float32),
    mesh=_mesh,
    compiler_params=pltpu.CompilerParams(needs_layout_passes=False),
    scratch_types=[
        pltpu.VMEM((3 * N_ATOMS,), jnp.float32),
        pltpu.VMEM((N_ATOMS,), jnp.int32),
        pltpu.VMEM((W_PAD,), jnp.float32),
        pltpu.VMEM((128,), jnp.float32),
        pltpu.VMEM((L,), jnp.float32),
        pltpu.VMEM((L,), jnp.float32),
    ],
)


def kernel(coordinates, atom_ix, weights, bias):
    xs = coordinates.T.reshape(-1)
    ai = atom_ix.astype(jnp.int32)
    wpad = jnp.pad(weights, (0, W_PAD - W_LEN))
    b16 = jnp.broadcast_to(bias.astype(jnp.float32), (L,))
    out16 = _sc_run(xs, ai, wpad, b16)
    return out16[:1]


# speedup vs baseline: 6414.4938x; 1.1166x over previous
import jax
import jax.numpy as jnp
from jax import lax
from jax.experimental import pallas as pl
from jax.experimental.pallas import tpu as pltpu
from jax.experimental.pallas import tpu_sc as plsc

N_ATOMS = 4096
N_ELEM = 118
W_LEN = N_ELEM * (N_ELEM + 1) // 2
W_PAD = 7040
COULOMB = -231000.0
L = 16
NSUB = 16
APT = N_ATOMS // NSUB


def _sc_body(xs_hbm, ai_hbm, w_hbm, bias_hbm, out_hbm,
             xs_v, ai_v, w_v, cnt_v, cnt2_v, comb_v, miscc_v, wpc_v,
             tmp_v, bias_v, out_v, cnt_sh, misc_sh, wp_sh):
    cid = lax.axis_index("c")
    sid = lax.axis_index("s")
    zero16 = jnp.zeros((L,), jnp.float32)
    ones16 = jnp.ones((L,), jnp.float32)
    iota = lax.iota(jnp.int32, L)

    @pl.when(cid == 0)
    def _():
        s = sid
        pltpu.sync_copy(ai_hbm.at[pl.ds(s * APT, APT)], ai_v)
        for c in range(3):
            pltpu.sync_copy(xs_hbm.at[pl.ds(c * N_ATOMS + s * APT, APT)],
                            xs_v.at[pl.ds(c * APT, APT)])
        pltpu.sync_copy(w_hbm, w_v)

        for k in range(8):
            cnt_v[pl.ds(k * L, L)] = zero16

        def hist_body(i, carry):
            idx = ai_v[pl.ds(i * L, L)]
            plsc.addupdate_scatter(cnt_v, [idx], ones16)
            return carry
        lax.fori_loop(0, APT // L, hist_body, 0)

        def comp_sums(c0):
            def body(i, carry):
                s1, sv = carry
                v = xs_v[pl.ds(c0 * APT + i * L, L)]
                return (s1 + v * v, sv + v)
            return lax.fori_loop(0, APT // L, body, (zero16, zero16))

        s1x, svx = comp_sums(0)
        s1y, svy = comp_sums(1)
        s1z, svz = comp_sums(2)
        s1 = jnp.sum(s1x + s1y + s1z)
        sx = jnp.sum(svx)
        sy = jnp.sum(svy)
        sz = jnp.sum(svz)
        misc = (jnp.where(iota == 0, s1, 0.0) + jnp.where(iota == 1, sx, 0.0)
                + jnp.where(iota == 2, sy, 0.0) + jnp.where(iota == 3, sz, 0.0))
        tmp_v[pl.ds(0, L)] = misc
        pltpu.sync_copy(cnt_v, cnt_sh.at[pl.ds(s * 128, 128)])
        pltpu.sync_copy(tmp_v, misc_sh.at[pl.ds(s * L, L)])

    plsc.subcore_barrier()

    @pl.when(cid == 0)
    def _():
        s = sid
        pltpu.sync_copy(cnt_sh, comb_v)
        for k in range(8):
            acc = zero16
            for r in range(NSUB):
                acc = acc + comb_v[pl.ds(r * 128 + k * L, L)]
            cnt2_v[pl.ds(k * L, L)] = acc

        wacc = zero16
        for k in range(8):
            a = s + NSUB * k
            t = a * (a + 1) // 2
            racc = zero16
            for kb in range(8):
                idx = jnp.minimum(t + kb * L + iota, W_PAD - 1)
                wv = plsc.load_gather(w_v, [idx])
                cb = cnt2_v[pl.ds(kb * L, L)]
                racc = racc + wv * cb
            cnta = plsc.load_gather(cnt2_v, [jnp.full((L,), 0, jnp.int32) + a])
            wacc = wacc + cnta * racc
        tmp_v[pl.ds(0, L)] = wacc
        pltpu.sync_copy(tmp_v, wp_sh.at[pl.ds(s * L, L)])

    plsc.subcore_barrier()

    @pl.when(jnp.logical_and(cid == 0, sid == 0))
    def _():
        pltpu.sync_copy(bias_hbm, bias_v)
        pltpu.sync_copy(wp_sh, wpc_v)
        pltpu.sync_copy(misc_sh, miscc_v)
        wv = zero16
        mv = zero16
        for r in range(NSUB):
            wv = wv + wpc_v[pl.ds(r * L, L)]
            mv = mv + miscc_v[pl.ds(r * L, L)]
        wsum = jnp.sum(wv)
        s1 = jnp.sum(jnp.where(iota == 0, mv, 0.0))
        sx = jnp.sum(jnp.where(iota == 1, mv, 0.0))
        sy = jnp.sum(jnp.where(iota == 2, mv, 0.0))
        sz = jnp.sum(jnp.where(iota == 3, mv, 0.0))
        d2 = 2.0 * N_ATOMS * s1 - 2.0 * (sx * sx + sy * sy + sz * sz)

        d2v = jnp.full((L,), d2)
        bits = lax.bitcast_convert_type(d2v, jnp.int32)
        y = lax.bitcast_convert_type(jnp.int32(0x5F3759DF) - (bits >> 1),
                                     jnp.float32)
        for _ in range(4):
            y = y * (1.5 - 0.5 * d2v * y * y)
        rd = jnp.where(d2v > 0.0, y, jnp.float32(3.4028235e38))

        bv = bias_v[pl.ds(0, L)]
        out_v[pl.ds(0, L)] = COULOMB * wsum * rd + bv
        pltpu.sync_copy(out_v, out_hbm)


_mesh = plsc.VectorSubcoreMesh(core_axis_name="c", subcore_axis_name="s")

_sc_run = pl.kernel(
    _sc_body,
    out_type=jax.ShapeDtypeStruct((L,), jnp.float32),
    mesh=_mesh,
    compiler_params=pltpu.CompilerParams(needs_layout_passes=False),
    scratch_types=[
        pltpu.VMEM((3 * APT,), jnp.float32),
        pltpu.VMEM((APT,), jnp.int32),
        pltpu.VMEM((W_PAD,), jnp.float32),
        pltpu.VMEM((128,), jnp.float32),
        pltpu.VMEM((128,), jnp.float32),
        pltpu.VMEM((NSUB * 128,), jnp.float32),
        pltpu.VMEM((NSUB * L,), jnp.float32),
        pltpu.VMEM((NSUB * L,), jnp.float32),
        pltpu.VMEM((L,), jnp.float32),
        pltpu.VMEM((L,), jnp.float32),
        pltpu.VMEM((L,), jnp.float32),
        pltpu.VMEM_SHARED((NSUB * 128,), jnp.float32),
        pltpu.VMEM_SHARED((NSUB * L,), jnp.float32),
        pltpu.VMEM_SHARED((NSUB * L,), jnp.float32),
    ],
)


def kernel(coordinates, atom_ix, weights, bias):
    xs = coordinates.T.reshape(-1)
    ai = atom_ix.astype(jnp.int32)
    wpad = jnp.pad(weights, (0, W_PAD - W_LEN))
    b16 = jnp.broadcast_to(bias.astype(jnp.float32), (L,))
    out16 = _sc_run(xs, ai, wpad, b16)
    return out16[:1]


# speedup vs baseline: 6456.9229x; 1.0066x over previous
import jax
import jax.numpy as jnp
from jax import lax
from jax.experimental import pallas as pl
from jax.experimental.pallas import tpu as pltpu
from jax.experimental.pallas import tpu_sc as plsc

N_ATOMS = 4096
N_ELEM = 118
W_LEN = N_ELEM * (N_ELEM + 1) // 2
W_PAD = 7040
COULOMB = -231000.0
L = 16
NSUB = 16
APT = N_ATOMS // NSUB
FPT = 3 * APT


def _sc_body(xs_hbm, ai_hbm, w_hbm, bias_hbm, out_hbm,
             xs_v, ai_v, w_v, cnt_v, cnt2_v, comb_v, miscc_v, wpc_v,
             tmp_v, bias_v, out_v, cnt_sh, misc_sh, wp_sh, wsem):
    cid = lax.axis_index("c")
    sid = lax.axis_index("s")
    zero16 = jnp.zeros((L,), jnp.float32)
    ones16 = jnp.ones((L,), jnp.float32)
    iota = lax.iota(jnp.int32, L)

    @pl.when(cid == 0)
    def _():
        s = sid
        wdma = pltpu.async_copy(w_hbm, w_v, wsem)
        pltpu.sync_copy(ai_hbm.at[pl.ds(s * APT, APT)], ai_v)
        pltpu.sync_copy(xs_hbm.at[pl.ds(s * FPT, FPT)], xs_v)

        for k in range(8):
            cnt_v[pl.ds(k * L, L)] = zero16

        def hist_body(i, carry):
            idx = ai_v[pl.ds(i * L, L)]
            plsc.addupdate_scatter(cnt_v, [idx], ones16)
            return carry
        lax.fori_loop(0, APT // L, hist_body, 0)

        masks = [[(iota + r) % 3 == c for c in range(3)] for r in range(3)]

        def coord_body(g, carry):
            s1, sxv, syv, szv = carry
            for r in range(3):
                v = xs_v[pl.ds((g * 3 + r) * L, L)]
                s1 = s1 + v * v
                sxv = sxv + jnp.where(masks[r][0], v, 0.0)
                syv = syv + jnp.where(masks[r][1], v, 0.0)
                szv = szv + jnp.where(masks[r][2], v, 0.0)
            return (s1, sxv, syv, szv)
        s1v, sxv, syv, szv = lax.fori_loop(
            0, FPT // (3 * L), coord_body, (zero16, zero16, zero16, zero16))

        s1 = jnp.sum(s1v)
        sx = jnp.sum(sxv)
        sy = jnp.sum(syv)
        sz = jnp.sum(szv)
        misc = (jnp.where(iota == 0, s1, 0.0) + jnp.where(iota == 1, sx, 0.0)
                + jnp.where(iota == 2, sy, 0.0) + jnp.where(iota == 3, sz, 0.0))
        tmp_v[pl.ds(0, L)] = misc
        pltpu.sync_copy(cnt_v, cnt_sh.at[pl.ds(s * 128, 128)])
        pltpu.sync_copy(tmp_v, misc_sh.at[pl.ds(s * L, L)])
        wdma.wait()

    plsc.subcore_barrier()

    @pl.when(cid == 0)
    def _():
        s = sid
        pltpu.sync_copy(cnt_sh, comb_v)
        for k in range(8):
            acc = zero16
            for r in range(NSUB):
                acc = acc + comb_v[pl.ds(r * 128 + k * L, L)]
            cnt2_v[pl.ds(k * L, L)] = acc

        wacc = zero16
        for k in range(8):
            a = s + NSUB * k
            t = a * (a + 1) // 2
            racc = zero16
            for kb in range(8):
                idx = jnp.minimum(t + kb * L + iota, W_PAD - 1)
                wv = plsc.load_gather(w_v, [idx])
                cb = cnt2_v[pl.ds(kb * L, L)]
                racc = racc + wv * cb
            cnta = plsc.load_gather(cnt2_v, [jnp.full((L,), 0, jnp.int32) + a])
            wacc = wacc + cnta * racc
        tmp_v[pl.ds(0, L)] = wacc
        pltpu.sync_copy(tmp_v, wp_sh.at[pl.ds(s * L, L)])

    plsc.subcore_barrier()

    @pl.when(jnp.logical_and(cid == 0, sid == 0))
    def _():
        pltpu.sync_copy(bias_hbm, bias_v)
        pltpu.sync_copy(wp_sh, wpc_v)
        pltpu.sync_copy(misc_sh, miscc_v)
        wv = zero16
        mv = zero16
        for r in range(NSUB):
            wv = wv + wpc_v[pl.ds(r * L, L)]
            mv = mv + miscc_v[pl.ds(r * L, L)]
        wsum = jnp.sum(wv)
        s1 = jnp.sum(jnp.where(iota == 0, mv, 0.0))
        sx = jnp.sum(jnp.where(iota == 1, mv, 0.0))
        sy = jnp.sum(jnp.where(iota == 2, mv, 0.0))
        sz = jnp.sum(jnp.where(iota == 3, mv, 0.0))
        d2 = 2.0 * N_ATOMS * s1 - 2.0 * (sx * sx + sy * sy + sz * sz)

        d2v = jnp.full((L,), d2)
        bits = lax.bitcast_convert_type(d2v, jnp.int32)
        y = lax.bitcast_convert_type(jnp.int32(0x5F3759DF) - (bits >> 1),
                                     jnp.float32)
        for _ in range(4):
            y = y * (1.5 - 0.5 * d2v * y * y)
        rd = jnp.where(d2v > 0.0, y, jnp.float32(3.4028235e38))

        bv = bias_v[pl.ds(0, L)]
        out_v[pl.ds(0, L)] = COULOMB * wsum * rd + bv
        pltpu.sync_copy(out_v, out_hbm)


_mesh = plsc.VectorSubcoreMesh(core_axis_name="c", subcore_axis_name="s")

_sc_run = pl.kernel(
    _sc_body,
    out_type=jax.ShapeDtypeStruct((L,), jnp.float32),
    mesh=_mesh,
    compiler_params=pltpu.CompilerParams(needs_layout_passes=False),
    scratch_types=[
        pltpu.VMEM((FPT,), jnp.float32),
        pltpu.VMEM((APT,), jnp.int32),
        pltpu.VMEM((W_PAD,), jnp.float32),
        pltpu.VMEM((128,), jnp.float32),
        pltpu.VMEM((128,), jnp.float32),
        pltpu.VMEM((NSUB * 128,), jnp.float32),
        pltpu.VMEM((NSUB * L,), jnp.float32),
        pltpu.VMEM((NSUB * L,), jnp.float32),
        pltpu.VMEM((L,), jnp.float32),
        pltpu.VMEM((L,), jnp.float32),
        pltpu.VMEM((L,), jnp.float32),
        pltpu.VMEM_SHARED((NSUB * 128,), jnp.float32),
        pltpu.VMEM_SHARED((NSUB * L,), jnp.float32),
        pltpu.VMEM_SHARED((NSUB * L,), jnp.float32),
        pltpu.SemaphoreType.DMA,
    ],
)


def kernel(coordinates, atom_ix, weights, bias):
    xs = coordinates.reshape(-1)
    ai = atom_ix.astype(jnp.int32)
    wpad = jnp.pad(weights, (0, W_PAD - W_LEN))
    b16 = jnp.broadcast_to(bias.astype(jnp.float32), (L,))
    out16 = _sc_run(xs, ai, wpad, b16)
    return out16[:1]
